# baseline (speedup 1.0000x reference)
import functools

import jax
import jax.numpy as jnp
from jax import lax
from jax.experimental import pallas as pl
from jax.experimental.pallas import tpu as pltpu
from jax.experimental.pallas import tpu_sc as plsc

NC = 2
NS = 16
L = 16
NW = NC * NS

EMB = 64
ROW = 128
CHUNK = 128
UBLK = 512


def _prep_block(uT_ref, iT_ref, ub_ref, ib_ref, uout_ref, iout_ref):
    eye = jnp.eye(EMB, dtype=jnp.float32)
    for src, bias, dst in ((uT_ref, ub_ref, uout_ref),
                           (iT_ref, ib_ref, iout_ref)):
        rows = lax.dot_general(src[...], eye, (((0,), (0,)), ((), ())),
                               preferred_element_type=jnp.float32)
        cat = jnp.concatenate(
            [rows, bias[...].reshape(UBLK, 1),
             jnp.zeros((UBLK, ROW - EMB - 1), jnp.float32)], axis=1)
        dst[...] = cat


def _make_prep(n_rows: int):
    nblk = n_rows // UBLK
    return pl.pallas_call(
        _prep_block,
        grid=(nblk,),
        compiler_params=pltpu.CompilerParams(
            fuse_transposed_lhs_in_matmul=True),
        in_specs=[
            pl.BlockSpec((EMB, UBLK), lambda j: (0, j)),
            pl.BlockSpec((EMB, UBLK), lambda j: (0, j)),
            pl.BlockSpec((UBLK,), lambda j: (j,)),
            pl.BlockSpec((UBLK,), lambda j: (j,)),
        ],
        out_specs=[
            pl.BlockSpec((UBLK, ROW), lambda j: (j, 0)),
            pl.BlockSpec((UBLK, ROW), lambda j: (j, 0)),
        ],
        out_shape=[
            jax.ShapeDtypeStruct((n_rows, ROW), jnp.float32),
            jax.ShapeDtypeStruct((n_rows, ROW), jnp.float32),
        ],
    )


def _make_sc_kernel(batch: int, n_rows: int):
    bpw = batch // NW
    nchunk = bpw // CHUNK
    blk_per_chunk = CHUNK // L

    mesh = plsc.VectorSubcoreMesh(core_axis_name="c", subcore_axis_name="s")

    @functools.partial(
        pl.kernel,
        mesh=mesh,
        compiler_params=pltpu.CompilerParams(needs_layout_passes=False),
        out_type=jax.ShapeDtypeStruct((batch,), jnp.float32),
        scratch_types=[
            pltpu.VMEM((nchunk, CHUNK), jnp.int32),
            pltpu.VMEM((nchunk, CHUNK), jnp.int32),
            pltpu.VMEM((CHUNK, ROW), jnp.float32),
            pltpu.VMEM((CHUNK, ROW), jnp.float32),
            pltpu.VMEM((CHUNK, ROW), jnp.float32),
            pltpu.VMEM((CHUNK, ROW), jnp.float32),
            pltpu.VMEM((L,), jnp.float32),
            pltpu.VMEM((bpw,), jnp.float32),
            pltpu.SemaphoreType.DMA,
            pltpu.SemaphoreType.DMA,
        ],
    )
    def sc_kernel(uid_hbm, iid_hbm, ucat_hbm, icat_hbm, mean_hbm, out_hbm,
                  uid_v, iid_v, ubuf0, ubuf1, ibuf0, ibuf1, mean_v, out_v,
                  sem0, sem1):
        ubufs = [ubuf0, ubuf1]
        ibufs = [ibuf0, ibuf1]
        sems = [sem0, sem1]
        wid = lax.axis_index("s") * NC + lax.axis_index("c")

        pltpu.sync_copy(uid_hbm.at[wid], uid_v)
        pltpu.sync_copy(iid_hbm.at[wid], iid_v)
        pltpu.sync_copy(mean_hbm, mean_v)

        def fire(j):
            b = j % 2
            cu = pltpu.async_copy(ucat_hbm.at[uid_v.at[j]], ubufs[b], sems[b])
            ci = pltpu.async_copy(icat_hbm.at[iid_v.at[j]], ibufs[b], sems[b])
            return (cu, ci)

        copies = [fire(0), fire(1)]

        mean_vec = mean_v[...]
        lanes = lax.iota(jnp.int32, L)
        bias_col = jnp.full((L,), EMB, jnp.int32)

        def make_blk_body(ub, ib, out_base):
            def blk_body(blk, carry):
                base = blk * L
                row = base + lanes
                accs = [
                    mean_vec
                    + plsc.load_gather(ub, [row, bias_col])
                    + plsc.load_gather(ib, [row, bias_col]),
                    jnp.zeros((L,), jnp.float32),
                    jnp.zeros((L,), jnp.float32),
                    jnp.zeros((L,), jnp.float32),
                ]
                for d in range(EMB):
                    col = jnp.full((L,), d, jnp.int32)
                    u = plsc.load_gather(ub, [row, col])
                    iv = plsc.load_gather(ib, [row, col])
                    accs[d % 4] = accs[d % 4] + u * iv
                out_v[pl.ds(out_base + base, L)] = (
                    (accs[0] + accs[1]) + (accs[2] + accs[3]))
                return carry
            return blk_body

        for j in range(nchunk):
            b = j % 2
            copies[j][0].wait()
            copies[j][1].wait()
            lax.fori_loop(0, blk_per_chunk,
                          make_blk_body(ubufs[b], ibufs[b], j * CHUNK), 0)
            if j + 2 < nchunk:
                copies.append(fire(j + 2))

        pltpu.sync_copy(out_v, out_hbm.at[pl.ds(wid * bpw, bpw)])

    return sc_kernel


def kernel(u_id, i_id, user_emb, user_bias, item_emb, item_bias,
           user_visual_emb, mean):
    batch = u_id.shape[0]
    nu = user_emb.shape[0]
    n_rows = ((nu + UBLK - 1) // UBLK) * UBLK

    uid3 = u_id.reshape(NW, batch // NW // CHUNK, CHUNK)
    iid3 = i_id.reshape(NW, batch // NW // CHUNK, CHUNK)
    ub = jnp.pad(user_bias.reshape(-1), (0, n_rows - nu))
    ib = jnp.pad(item_bias.reshape(-1), (0, n_rows - nu))
    mean_l = jnp.broadcast_to(mean, (L,))

    ucat, icat = _make_prep(n_rows)(user_emb.T, item_emb.T, ub, ib)

    sc = _make_sc_kernel(batch, n_rows)
    return sc(uid3, iid3, ucat, icat, mean_l)


# speedup vs baseline: 1.5092x; 1.5092x over previous
import functools

import jax
import jax.numpy as jnp
from jax import lax
from jax.experimental import pallas as pl
from jax.experimental.pallas import tpu as pltpu
from jax.experimental.pallas import tpu_sc as plsc

NC = 2
NS = 16
L = 16
NW = NC * NS

EMB = 64
ROW = 128
CHUNK = 128
UBLK = 2048


def _prep_block(uT_ref, iT_ref, ub_ref, ib_ref, uout_ref, iout_ref):
    eye = jnp.eye(EMB, dtype=jnp.float32)
    for src, bias, dst in ((uT_ref, ub_ref, uout_ref),
                           (iT_ref, ib_ref, iout_ref)):
        rows = lax.dot_general(src[...], eye, (((0,), (0,)), ((), ())),
                               preferred_element_type=jnp.float32)
        cat = jnp.concatenate(
            [rows, bias[...].reshape(UBLK, 1),
             jnp.zeros((UBLK, ROW - EMB - 1), jnp.float32)], axis=1)
        dst[...] = cat


def _make_prep(n_rows: int):
    nblk = n_rows // UBLK
    return pl.pallas_call(
        _prep_block,
        grid=(nblk,),
        compiler_params=pltpu.CompilerParams(
            fuse_transposed_lhs_in_matmul=True),
        in_specs=[
            pl.BlockSpec((EMB, UBLK), lambda j: (0, j)),
            pl.BlockSpec((EMB, UBLK), lambda j: (0, j)),
            pl.BlockSpec((UBLK,), lambda j: (j,)),
            pl.BlockSpec((UBLK,), lambda j: (j,)),
        ],
        out_specs=[
            pl.BlockSpec((UBLK, ROW), lambda j: (j, 0)),
            pl.BlockSpec((UBLK, ROW), lambda j: (j, 0)),
        ],
        out_shape=[
            jax.ShapeDtypeStruct((n_rows, ROW), jnp.float32),
            jax.ShapeDtypeStruct((n_rows, ROW), jnp.float32),
        ],
    )


def _make_sc_kernel(batch: int, n_rows: int):
    bpw = batch // NW
    nchunk = bpw // CHUNK
    blk_per_chunk = CHUNK // L

    mesh = plsc.VectorSubcoreMesh(core_axis_name="c", subcore_axis_name="s")

    @functools.partial(
        pl.kernel,
        mesh=mesh,
        compiler_params=pltpu.CompilerParams(needs_layout_passes=False),
        out_type=jax.ShapeDtypeStruct((batch,), jnp.float32),
        scratch_types=[
            pltpu.VMEM((nchunk, CHUNK), jnp.int32),
            pltpu.VMEM((nchunk, CHUNK), jnp.int32),
            pltpu.VMEM((CHUNK, ROW), jnp.float32),
            pltpu.VMEM((CHUNK, ROW), jnp.float32),
            pltpu.VMEM((CHUNK, ROW), jnp.float32),
            pltpu.VMEM((CHUNK, ROW), jnp.float32),
            pltpu.VMEM((L,), jnp.float32),
            pltpu.VMEM((bpw,), jnp.float32),
            pltpu.SemaphoreType.DMA,
            pltpu.SemaphoreType.DMA,
        ],
    )
    def sc_kernel(uid_hbm, iid_hbm, ucat_hbm, icat_hbm, mean_hbm, out_hbm,
                  uid_v, iid_v, ubuf0, ubuf1, ibuf0, ibuf1, mean_v, out_v,
                  sem0, sem1):
        ubufs = [ubuf0, ubuf1]
        ibufs = [ibuf0, ibuf1]
        sems = [sem0, sem1]
        wid = lax.axis_index("s") * NC + lax.axis_index("c")

        pltpu.sync_copy(uid_hbm.at[wid], uid_v)
        pltpu.sync_copy(iid_hbm.at[wid], iid_v)
        pltpu.sync_copy(mean_hbm, mean_v)

        def fire(j):
            b = j % 2
            cu = pltpu.async_copy(ucat_hbm.at[uid_v.at[j]], ubufs[b], sems[b])
            ci = pltpu.async_copy(icat_hbm.at[iid_v.at[j]], ibufs[b], sems[b])
            return (cu, ci)

        copies = [fire(0), fire(1)]

        mean_vec = mean_v[...]
        lanes = lax.iota(jnp.int32, L)
        bias_col = jnp.full((L,), EMB, jnp.int32)

        def make_blk_body(ub, ib, out_base):
            def blk_body(blk, carry):
                base = blk * L
                row = base + lanes
                accs = [
                    mean_vec
                    + plsc.load_gather(ub, [row, bias_col])
                    + plsc.load_gather(ib, [row, bias_col]),
                    jnp.zeros((L,), jnp.float32),
                    jnp.zeros((L,), jnp.float32),
                    jnp.zeros((L,), jnp.float32),
                ]
                for d in range(EMB):
                    col = jnp.full((L,), d, jnp.int32)
                    u = plsc.load_gather(ub, [row, col])
                    iv = plsc.load_gather(ib, [row, col])
                    accs[d % 4] = accs[d % 4] + u * iv
                out_v[pl.ds(out_base + base, L)] = (
                    (accs[0] + accs[1]) + (accs[2] + accs[3]))
                return carry
            return blk_body

        for j in range(nchunk):
            b = j % 2
            copies[j][0].wait()
            copies[j][1].wait()
            lax.fori_loop(0, blk_per_chunk,
                          make_blk_body(ubufs[b], ibufs[b], j * CHUNK), 0)
            if j + 2 < nchunk:
                copies.append(fire(j + 2))

        pltpu.sync_copy(out_v, out_hbm.at[pl.ds(wid * bpw, bpw)])

    return sc_kernel


def kernel(u_id, i_id, user_emb, user_bias, item_emb, item_bias,
           user_visual_emb, mean):
    batch = u_id.shape[0]
    nu = user_emb.shape[0]
    n_rows = ((nu + UBLK - 1) // UBLK) * UBLK

    uid3 = u_id.reshape(NW, batch // NW // CHUNK, CHUNK)
    iid3 = i_id.reshape(NW, batch // NW // CHUNK, CHUNK)
    ub = jnp.pad(user_bias.reshape(-1), (0, n_rows - nu))
    ib = jnp.pad(item_bias.reshape(-1), (0, n_rows - nu))
    mean_l = jnp.broadcast_to(mean, (L,))

    ucat, icat = _make_prep(n_rows)(user_emb.T, item_emb.T, ub, ib)

    sc = _make_sc_kernel(batch, n_rows)
    return sc(uid3, iid3, ucat, icat, mean_l)


# speedup vs baseline: 1.7170x; 1.1377x over previous
import functools

import jax
import jax.numpy as jnp
from jax import lax
from jax.experimental import pallas as pl
from jax.experimental.pallas import tpu as pltpu
from jax.experimental.pallas import tpu_sc as plsc

NC = 2
NS = 16
L = 16
NW = NC * NS

EMB = 64
ROW = 128
CHUNK = 128
UBLK = 8192


def _prep_block(uT_ref, iT_ref, ub_ref, ib_ref, uout_ref, iout_ref):
    eye = jnp.eye(EMB, dtype=jnp.float32)
    for src, bias, dst in ((uT_ref, ub_ref, uout_ref),
                           (iT_ref, ib_ref, iout_ref)):
        rows = lax.dot_general(src[...], eye, (((0,), (0,)), ((), ())),
                               preferred_element_type=jnp.float32)
        cat = jnp.concatenate(
            [rows, bias[...].reshape(UBLK, 1),
             jnp.zeros((UBLK, ROW - EMB - 1), jnp.float32)], axis=1)
        dst[...] = cat


def _make_prep(n_rows: int):
    nblk = n_rows // UBLK
    return pl.pallas_call(
        _prep_block,
        grid=(nblk,),
        compiler_params=pltpu.CompilerParams(
            fuse_transposed_lhs_in_matmul=True),
        in_specs=[
            pl.BlockSpec((EMB, UBLK), lambda j: (0, j)),
            pl.BlockSpec((EMB, UBLK), lambda j: (0, j)),
            pl.BlockSpec((UBLK,), lambda j: (j,)),
            pl.BlockSpec((UBLK,), lambda j: (j,)),
        ],
        out_specs=[
            pl.BlockSpec((UBLK, ROW), lambda j: (j, 0)),
            pl.BlockSpec((UBLK, ROW), lambda j: (j, 0)),
        ],
        out_shape=[
            jax.ShapeDtypeStruct((n_rows, ROW), jnp.float32),
            jax.ShapeDtypeStruct((n_rows, ROW), jnp.float32),
        ],
    )


def _make_sc_kernel(batch: int, n_rows: int):
    bpw = batch // NW
    nchunk = bpw // CHUNK
    blk_per_chunk = CHUNK // L

    mesh = plsc.VectorSubcoreMesh(core_axis_name="c", subcore_axis_name="s")

    @functools.partial(
        pl.kernel,
        mesh=mesh,
        compiler_params=pltpu.CompilerParams(needs_layout_passes=False),
        out_type=jax.ShapeDtypeStruct((batch,), jnp.float32),
        scratch_types=[
            pltpu.VMEM((nchunk, CHUNK), jnp.int32),
            pltpu.VMEM((nchunk, CHUNK), jnp.int32),
            pltpu.VMEM((CHUNK, ROW), jnp.float32),
            pltpu.VMEM((CHUNK, ROW), jnp.float32),
            pltpu.VMEM((CHUNK, ROW), jnp.float32),
            pltpu.VMEM((CHUNK, ROW), jnp.float32),
            pltpu.VMEM((L,), jnp.float32),
            pltpu.VMEM((bpw,), jnp.float32),
            pltpu.SemaphoreType.DMA,
            pltpu.SemaphoreType.DMA,
        ],
    )
    def sc_kernel(uid_hbm, iid_hbm, ucat_hbm, icat_hbm, mean_hbm, out_hbm,
                  uid_v, iid_v, ubuf0, ubuf1, ibuf0, ibuf1, mean_v, out_v,
                  sem0, sem1):
        ubufs = [ubuf0, ubuf1]
        ibufs = [ibuf0, ibuf1]
        sems = [sem0, sem1]
        wid = lax.axis_index("s") * NC + lax.axis_index("c")

        pltpu.sync_copy(uid_hbm.at[wid], uid_v)
        pltpu.sync_copy(iid_hbm.at[wid], iid_v)
        pltpu.sync_copy(mean_hbm, mean_v)

        def fire(j):
            b = j % 2
            cu = pltpu.async_copy(ucat_hbm.at[uid_v.at[j]], ubufs[b], sems[b])
            ci = pltpu.async_copy(icat_hbm.at[iid_v.at[j]], ibufs[b], sems[b])
            return (cu, ci)

        copies = [fire(0), fire(1)]

        mean_vec = mean_v[...]
        lanes = lax.iota(jnp.int32, L)
        bias_col = jnp.full((L,), EMB, jnp.int32)

        def make_blk_body(ub, ib, out_base):
            def blk_body(blk, carry):
                base = blk * L
                row = base + lanes
                accs = [
                    mean_vec
                    + plsc.load_gather(ub, [row, bias_col])
                    + plsc.load_gather(ib, [row, bias_col]),
                    jnp.zeros((L,), jnp.float32),
                    jnp.zeros((L,), jnp.float32),
                    jnp.zeros((L,), jnp.float32),
                ]
                for d in range(EMB):
                    col = jnp.full((L,), d, jnp.int32)
                    u = plsc.load_gather(ub, [row, col])
                    iv = plsc.load_gather(ib, [row, col])
                    accs[d % 4] = accs[d % 4] + u * iv
                out_v[pl.ds(out_base + base, L)] = (
                    (accs[0] + accs[1]) + (accs[2] + accs[3]))
                return carry
            return blk_body

        for j in range(nchunk):
            b = j % 2
            copies[j][0].wait()
            copies[j][1].wait()
            lax.fori_loop(0, blk_per_chunk,
                          make_blk_body(ubufs[b], ibufs[b], j * CHUNK), 0)
            if j + 2 < nchunk:
                copies.append(fire(j + 2))

        pltpu.sync_copy(out_v, out_hbm.at[pl.ds(wid * bpw, bpw)])

    return sc_kernel


def kernel(u_id, i_id, user_emb, user_bias, item_emb, item_bias,
           user_visual_emb, mean):
    batch = u_id.shape[0]
    nu = user_emb.shape[0]
    n_rows = ((nu + UBLK - 1) // UBLK) * UBLK

    uid3 = u_id.reshape(NW, batch // NW // CHUNK, CHUNK)
    iid3 = i_id.reshape(NW, batch // NW // CHUNK, CHUNK)
    ub = jnp.pad(user_bias.reshape(-1), (0, n_rows - nu))
    ib = jnp.pad(item_bias.reshape(-1), (0, n_rows - nu))
    mean_l = jnp.broadcast_to(mean, (L,))

    ucat, icat = _make_prep(n_rows)(user_emb.T, item_emb.T, ub, ib)

    sc = _make_sc_kernel(batch, n_rows)
    return sc(uid3, iid3, ucat, icat, mean_l)
